# Initial kernel scaffold; baseline (speedup 1.0000x reference)
import jax
import jax.numpy as jnp
from jax.experimental import pallas as pl


def kernel(attn_s):
    raise NotImplementedError("write your pallas kernel here")


# speedup vs baseline: 5.1228x; 5.1228x over previous
import functools

import jax
import jax.numpy as jnp
from jax import lax
from jax.experimental import pallas as pl
from jax.experimental.pallas import tpu as pltpu
from jax.experimental.pallas import tpu_sc as plsc

ROWS = 128
COLS = 32768
K_SEL = 32
EPS = 1e-7

_info = plsc.get_sparse_core_info()
NC = _info.num_cores
NS = _info.num_subcores
L = _info.num_lanes
NW = NC * NS
RPW = ROWS // NW
NVREG = COLS // L
HIST_VREGS = 257
UNROLL = 8

_PASSES = ((31, 23, 8), (23, 15, 8), (15, 7, 8), (7, 0, 7))


def _build():
    mesh = plsc.VectorSubcoreMesh(core_axis_name="c", subcore_axis_name="s")

    @functools.partial(
        pl.kernel,
        mesh=mesh,
        out_type=jax.ShapeDtypeStruct((ROWS, COLS), jnp.float32),
        scratch_types=[
            pltpu.VMEM((COLS,), jnp.float32),
            pltpu.VMEM((HIST_VREGS * L,), jnp.int32),
        ],
        compiler_params=pltpu.CompilerParams(needs_layout_passes=False),
    )
    def sc_topk_norm(in_hbm, out_hbm, row_v, hist_v):
        wid = lax.axis_index("s") * NC + lax.axis_index("c")
        lane = jnp.arange(L, dtype=jnp.int32)
        ones = jnp.ones((L,), jnp.int32)

        def do_row(r, carry):
            row = wid * RPW + r
            pltpu.sync_copy(in_hbm.at[row], row_v)

            prefix = jnp.uint32(0)
            rank = jnp.int32(K_SEL)

            for (m_shift, d_shift, width) in _PASSES:
                def zero_body(d, _):
                    hist_v[pl.ds(d * L, L)] = jnp.zeros((L,), jnp.int32)
                    return 0
                lax.fori_loop(0, HIST_VREGS, zero_body, 0)

                def hist_body(jj, pfx):
                    for u in range(UNROLL):
                        j = jj * UNROLL + u
                        v = row_v[pl.ds(j * L, L)]
                        bits = lax.bitcast_convert_type(v, jnp.uint32)
                        match = (bits >> m_shift) == pfx
                        digit = ((bits >> d_shift) & jnp.uint32(0xFF)).astype(jnp.int32)
                        idx = (digit << 4) | lane
                        plsc.addupdate_scatter(hist_v, [idx], ones, mask=match)
                    return pfx
                lax.fori_loop(0, NVREG // UNROLL, hist_body, prefix)

                def suf_body(i, acc):
                    d = 255 - i
                    acc = acc + hist_v[pl.ds(d * L, L)]
                    hist_v[pl.ds(d * L, L)] = acc
                    return acc
                lax.fori_loop(0, 256, suf_body, jnp.zeros((L,), jnp.int32))

                def bs_body(_, lohi):
                    lo, hi = lohi
                    mid = (lo + hi) // 2
                    c = jnp.sum(hist_v[pl.ds(mid * L, L)])
                    ge = c >= rank
                    return (jnp.where(ge, mid, lo), jnp.where(ge, hi, mid))
                b, _ = lax.fori_loop(0, 8, bs_body,
                                     (jnp.int32(0), jnp.int32(256)))

                above = jnp.sum(hist_v[pl.ds((b + 1) * L, L)])
                rank = rank - above
                prefix = (prefix << jnp.uint32(width)) | b.astype(jnp.uint32)

            v32 = lax.bitcast_convert_type(jnp.broadcast_to(prefix, (L,)),
                                           jnp.float32)
            delta = v32 + jnp.float32(EPS)

            def sum_body(jj, acc):
                for u in range(UNROLL):
                    j = jj * UNROLL + u
                    v = row_v[pl.ds(j * L, L)]
                    acc = acc + jnp.maximum(v - delta, jnp.float32(0.0))
                return acc
            acc = lax.fori_loop(0, NVREG // UNROLL, sum_body,
                                jnp.zeros((L,), jnp.float32))
            s_total = jnp.sum(acc) + jnp.float32(EPS)
            rinv = jnp.full((L,), 1.0, jnp.float32) / jnp.broadcast_to(
                s_total, (L,))

            def norm_body(jj, _):
                for u in range(UNROLL):
                    j = jj * UNROLL + u
                    v = row_v[pl.ds(j * L, L)]
                    w = jnp.maximum(v - delta, jnp.float32(0.0))
                    row_v[pl.ds(j * L, L)] = w * rinv
                return 0
            lax.fori_loop(0, NVREG // UNROLL, norm_body, 0)

            pltpu.sync_copy(row_v, out_hbm.at[row])
            return carry

        lax.fori_loop(0, RPW, do_row, 0)

    return sc_topk_norm


_sc_kernel = _build()


def kernel(attn_s):
    return _sc_kernel(attn_s)


# speedup vs baseline: 15.5240x; 3.0304x over previous
import functools

import jax
import jax.numpy as jnp
from jax import lax
from jax.experimental import pallas as pl
from jax.experimental.pallas import tpu as pltpu
from jax.experimental.pallas import tpu_sc as plsc

ROWS = 128
COLS = 32768
K_SEL = 32
EPS = 1e-7

_info = plsc.get_sparse_core_info()
NC = _info.num_cores
NS = _info.num_subcores
L = _info.num_lanes
NW = NC * NS
RPW = ROWS // NW
NVREG = COLS // L
HIST_VREGS = 257
UNROLL = 8

_PASSES = ((31, 23, 8), (23, 15, 8), (15, 7, 8), (7, 0, 7))


def _build():
    mesh = plsc.VectorSubcoreMesh(core_axis_name="c", subcore_axis_name="s")

    @functools.partial(
        pl.kernel,
        mesh=mesh,
        out_type=jax.ShapeDtypeStruct((ROWS, COLS), jnp.float32),
        scratch_types=[
            pltpu.VMEM((COLS,), jnp.float32),
            pltpu.VMEM((HIST_VREGS * L,), jnp.int32),
        ],
        compiler_params=pltpu.CompilerParams(needs_layout_passes=False),
    )
    def sc_topk_norm(in_hbm, out_hbm, row_v, hist_v):
        wid = lax.axis_index("s") * NC + lax.axis_index("c")
        lane = jnp.arange(L, dtype=jnp.int32)
        ones = jnp.ones((L,), jnp.int32)

        def do_row(r, carry):
            row = wid * RPW + r
            pltpu.sync_copy(in_hbm.at[row], row_v)

            prefix = jnp.uint32(0)
            rank = jnp.int32(K_SEL)

            for (m_shift, d_shift, width) in _PASSES:
                @plsc.parallel_loop(0, HIST_VREGS * L, L, unroll=4)
                def _(off):
                    hist_v[pl.ds(off, L)] = jnp.zeros((L,), jnp.int32)

                pfx = prefix

                @plsc.parallel_loop(0, COLS, L, unroll=UNROLL)
                def _(off):
                    v = row_v[pl.ds(off, L)]
                    bits = lax.bitcast_convert_type(v, jnp.uint32)
                    match = (bits >> m_shift) == pfx
                    digit = ((bits >> d_shift) & jnp.uint32(0xFF)).astype(jnp.int32)
                    idx = (digit << 4) | lane
                    plsc.addupdate_scatter(hist_v, [idx], ones, mask=match)

                def suf_body(i, acc):
                    d = 255 - i
                    acc = acc + hist_v[pl.ds(d * L, L)]
                    hist_v[pl.ds(d * L, L)] = acc
                    return acc
                lax.fori_loop(0, 256, suf_body, jnp.zeros((L,), jnp.int32))

                def bs_body(_, lohi):
                    lo, hi = lohi
                    mid = (lo + hi) // 2
                    c = jnp.sum(hist_v[pl.ds(mid * L, L)])
                    ge = c >= rank
                    return (jnp.where(ge, mid, lo), jnp.where(ge, hi, mid))
                b, _ = lax.fori_loop(0, 8, bs_body,
                                     (jnp.int32(0), jnp.int32(256)))

                above = jnp.sum(hist_v[pl.ds((b + 1) * L, L)])
                rank = rank - above
                prefix = (prefix << jnp.uint32(width)) | b.astype(jnp.uint32)

            v32 = lax.bitcast_convert_type(jnp.broadcast_to(prefix, (L,)),
                                           jnp.float32)
            delta = v32 + jnp.float32(EPS)

            @plsc.parallel_loop(0, COLS, L, unroll=UNROLL,
                                carry=jnp.zeros((L,), jnp.float32))
            def acc(off, a):
                v = row_v[pl.ds(off, L)]
                return a + jnp.maximum(v - delta, jnp.float32(0.0))
            s_total = jnp.sum(acc) + jnp.float32(EPS)
            rinv = jnp.full((L,), 1.0, jnp.float32) / jnp.broadcast_to(
                s_total, (L,))

            @plsc.parallel_loop(0, COLS, L, unroll=UNROLL)
            def _(off):
                v = row_v[pl.ds(off, L)]
                w = jnp.maximum(v - delta, jnp.float32(0.0))
                row_v[pl.ds(off, L)] = w * rinv

            pltpu.sync_copy(row_v, out_hbm.at[row])
            return carry

        lax.fori_loop(0, RPW, do_row, 0)

    return sc_topk_norm


_sc_kernel = _build()


def kernel(attn_s):
    return _sc_kernel(attn_s)


# speedup vs baseline: 16.0909x; 1.0365x over previous
import functools

import jax
import jax.numpy as jnp
from jax import lax
from jax.experimental import pallas as pl
from jax.experimental.pallas import tpu as pltpu
from jax.experimental.pallas import tpu_sc as plsc

ROWS = 128
COLS = 32768
K_SEL = 32
EPS = 1e-7

_info = plsc.get_sparse_core_info()
NC = _info.num_cores
NS = _info.num_subcores
L = _info.num_lanes
NW = NC * NS
RPW = ROWS // NW
NVREG = COLS // L
HIST_VREGS = 257
UNROLL = 16

_PASSES = ((31, 23, 8), (23, 15, 8), (15, 7, 8), (7, 0, 7))


def _build():
    mesh = plsc.VectorSubcoreMesh(core_axis_name="c", subcore_axis_name="s")

    @functools.partial(
        pl.kernel,
        mesh=mesh,
        out_type=jax.ShapeDtypeStruct((ROWS, COLS), jnp.float32),
        scratch_types=[
            pltpu.VMEM((COLS,), jnp.float32),
            pltpu.VMEM((HIST_VREGS * L,), jnp.int32),
        ],
        compiler_params=pltpu.CompilerParams(needs_layout_passes=False),
    )
    def sc_topk_norm(in_hbm, out_hbm, row_v, hist_v):
        wid = lax.axis_index("s") * NC + lax.axis_index("c")
        lane_u = jnp.arange(L, dtype=jnp.uint32)
        ones = jnp.ones((L,), jnp.int32)

        def do_row(r, carry):
            row = wid * RPW + r
            pltpu.sync_copy(in_hbm.at[row], row_v)

            prefix = jnp.uint32(0)
            rank = jnp.int32(K_SEL)

            for (m_shift, d_shift, width) in _PASSES:
                @plsc.parallel_loop(0, HIST_VREGS * L, L, unroll=4)
                def _(off):
                    hist_v[pl.ds(off, L)] = jnp.zeros((L,), jnp.int32)

                pfx = prefix
                idx_mask = jnp.uint32(0xFF0 if d_shift >= 4 else 0x7F0)
                first = m_shift == 31

                @plsc.parallel_loop(0, COLS, L, unroll=UNROLL)
                def _(off):
                    v = row_v[pl.ds(off, L)]
                    bits = lax.bitcast_convert_type(v, jnp.uint32)
                    if d_shift >= 4:
                        sh = (bits >> (d_shift - 4)) & idx_mask
                    else:
                        sh = (bits << (4 - d_shift)) & idx_mask
                    idx = (sh | lane_u).astype(jnp.int32)
                    if first:
                        plsc.addupdate_scatter(hist_v, [idx], ones)
                    else:
                        match = (bits >> m_shift) == pfx
                        plsc.addupdate_scatter(hist_v, [idx], ones, mask=match)

                def suf_body(i, acc):
                    d = 255 - i
                    acc = acc + hist_v[pl.ds(d * L, L)]
                    hist_v[pl.ds(d * L, L)] = acc
                    return acc
                lax.fori_loop(0, 256, suf_body, jnp.zeros((L,), jnp.int32))

                def bs_body(_, lohi):
                    lo, hi = lohi
                    mid = (lo + hi) // 2
                    c = jnp.sum(hist_v[pl.ds(mid * L, L)])
                    ge = c >= rank
                    return (jnp.where(ge, mid, lo), jnp.where(ge, hi, mid))
                b, _ = lax.fori_loop(0, 8, bs_body,
                                     (jnp.int32(0), jnp.int32(256)))

                above = jnp.sum(hist_v[pl.ds((b + 1) * L, L)])
                rank = rank - above
                prefix = (prefix << jnp.uint32(width)) | b.astype(jnp.uint32)

            v32 = lax.bitcast_convert_type(jnp.broadcast_to(prefix, (L,)),
                                           jnp.float32)
            delta = v32 + jnp.float32(EPS)

            zero = jnp.float32(0.0)

            @plsc.parallel_loop(0, COLS, 8 * L, unroll=2,
                                carry=jnp.zeros((L,), jnp.float32))
            def acc(off, a):
                ws = [jnp.maximum(row_v[pl.ds(off + k * L, L)] - delta, zero)
                      for k in range(8)]
                t01, t23 = ws[0] + ws[1], ws[2] + ws[3]
                t45, t67 = ws[4] + ws[5], ws[6] + ws[7]
                return a + ((t01 + t23) + (t45 + t67))
            s_total = jnp.sum(acc) + jnp.float32(EPS)
            rinv = jnp.full((L,), 1.0, jnp.float32) / jnp.broadcast_to(
                s_total, (L,))

            @plsc.parallel_loop(0, COLS, L, unroll=UNROLL)
            def _(off):
                v = row_v[pl.ds(off, L)]
                w = jnp.maximum(v - delta, jnp.float32(0.0))
                row_v[pl.ds(off, L)] = w * rinv

            pltpu.sync_copy(row_v, out_hbm.at[row])
            return carry

        lax.fori_loop(0, RPW, do_row, 0)

    return sc_topk_norm


_sc_kernel = _build()


def kernel(attn_s):
    return _sc_kernel(attn_s)


# speedup vs baseline: 16.5038x; 1.0257x over previous
import functools

import jax
import jax.numpy as jnp
from jax import lax
from jax.experimental import pallas as pl
from jax.experimental.pallas import tpu as pltpu
from jax.experimental.pallas import tpu_sc as plsc

ROWS = 128
COLS = 32768
K_SEL = 32
EPS = 1e-7

_info = plsc.get_sparse_core_info()
NC = _info.num_cores
NS = _info.num_subcores
L = _info.num_lanes
NW = NC * NS
RPW = ROWS // NW
NVREG = COLS // L
HIST_VREGS = 257
UNROLL = 16

_PASSES = ((31, 23, 8), (23, 15, 8), (15, 7, 8), (7, 0, 7))


def _build():
    mesh = plsc.VectorSubcoreMesh(core_axis_name="c", subcore_axis_name="s")

    @functools.partial(
        pl.kernel,
        mesh=mesh,
        out_type=jax.ShapeDtypeStruct((ROWS, COLS), jnp.float32),
        scratch_types=[
            pltpu.VMEM((COLS,), jnp.float32),
            pltpu.VMEM((COLS,), jnp.float32),
            pltpu.VMEM((HIST_VREGS * L,), jnp.int32),
            pltpu.SemaphoreType.DMA,
            pltpu.SemaphoreType.DMA,
        ],
        compiler_params=pltpu.CompilerParams(needs_layout_passes=False),
    )
    def sc_topk_norm(in_hbm, out_hbm, buf0, buf1, hist_v, sem0, sem1):
        wid = lax.axis_index("s") * NC + lax.axis_index("c")
        base = wid * RPW
        lane_u = jnp.arange(L, dtype=jnp.uint32)
        ones = jnp.ones((L,), jnp.int32)

        def process(row_v):
            prefix = jnp.uint32(0)
            rank = jnp.int32(K_SEL)

            for (m_shift, d_shift, width) in _PASSES:
                @plsc.parallel_loop(0, HIST_VREGS * L, L, unroll=4)
                def _(off):
                    hist_v[pl.ds(off, L)] = jnp.zeros((L,), jnp.int32)

                pfx = prefix
                idx_mask = jnp.uint32(0xFF0 if d_shift >= 4 else 0x7F0)
                first = m_shift == 31

                @plsc.parallel_loop(0, COLS, L, unroll=UNROLL)
                def _(off):
                    v = row_v[pl.ds(off, L)]
                    bits = lax.bitcast_convert_type(v, jnp.uint32)
                    if d_shift >= 4:
                        sh = (bits >> (d_shift - 4)) & idx_mask
                    else:
                        sh = (bits << (4 - d_shift)) & idx_mask
                    idx = (sh | lane_u).astype(jnp.int32)
                    if first:
                        plsc.addupdate_scatter(hist_v, [idx], ones)
                    else:
                        match = (bits >> m_shift) == pfx
                        plsc.addupdate_scatter(hist_v, [idx], ones, mask=match)

                def suf_body(i, acc):
                    d = 255 - i
                    acc = acc + hist_v[pl.ds(d * L, L)]
                    hist_v[pl.ds(d * L, L)] = acc
                    return acc
                lax.fori_loop(0, 256, suf_body, jnp.zeros((L,), jnp.int32))

                def bs_body(_, lohi):
                    lo, hi = lohi
                    mid = (lo + hi) // 2
                    c = jnp.sum(hist_v[pl.ds(mid * L, L)])
                    ge = c >= rank
                    return (jnp.where(ge, mid, lo), jnp.where(ge, hi, mid))
                b, _ = lax.fori_loop(0, 8, bs_body,
                                     (jnp.int32(0), jnp.int32(256)))

                above = jnp.sum(hist_v[pl.ds((b + 1) * L, L)])
                rank = rank - above
                prefix = (prefix << jnp.uint32(width)) | b.astype(jnp.uint32)

            v32 = lax.bitcast_convert_type(jnp.broadcast_to(prefix, (L,)),
                                           jnp.float32)
            delta = v32 + jnp.float32(EPS)

            zero = jnp.float32(0.0)

            @plsc.parallel_loop(0, COLS, 8 * L, unroll=2,
                                carry=jnp.zeros((L,), jnp.float32))
            def acc(off, a):
                ws = [jnp.maximum(row_v[pl.ds(off + k * L, L)] - delta, zero)
                      for k in range(8)]
                t01, t23 = ws[0] + ws[1], ws[2] + ws[3]
                t45, t67 = ws[4] + ws[5], ws[6] + ws[7]
                return a + ((t01 + t23) + (t45 + t67))
            s_total = jnp.sum(acc) + jnp.float32(EPS)
            rinv = jnp.full((L,), 1.0, jnp.float32) / jnp.broadcast_to(
                s_total, (L,))

            @plsc.parallel_loop(0, COLS, L, unroll=UNROLL)
            def _(off):
                v = row_v[pl.ds(off, L)]
                w = jnp.maximum(v - delta, jnp.float32(0.0))
                row_v[pl.ds(off, L)] = w * rinv

        bufs = (buf0, buf1)
        sems = (sem0, sem1)
        loads = [None] * RPW
        stores = [None] * RPW
        loads[0] = pltpu.async_copy(in_hbm.at[base], buf0, sem0)
        loads[1] = pltpu.async_copy(in_hbm.at[base + 1], buf1, sem1)
        for r in range(RPW):
            b = r % 2
            loads[r].wait()
            process(bufs[b])
            stores[r] = pltpu.async_copy(bufs[b], out_hbm.at[base + r],
                                         sems[b])
            if r + 2 < RPW:
                stores[r].wait()
                loads[r + 2] = pltpu.async_copy(in_hbm.at[base + r + 2],
                                                bufs[b], sems[b])
        stores[RPW - 2].wait()
        stores[RPW - 1].wait()

    return sc_topk_norm


_sc_kernel = _build()


def kernel(attn_s):
    return _sc_kernel(attn_s)
